# baseline (speedup 1.0000x reference)
import functools

import jax
import jax.numpy as jnp
from jax import lax
from jax.experimental import pallas as pl
from jax.experimental.pallas import tpu as pltpu
from jax.experimental.pallas import tpu_sc as plsc

MAX_PT = 1000000
EMBED_DIM = 32
BATCH = 16384

NC = 2
NS = 16
NW = NC * NS
B_PER_W = BATCH // NW
CHUNK = 128
N_CHUNK = B_PER_W // CHUNK
NQ = 4
QDIM = EMBED_DIM // NQ

_mesh = plsc.VectorSubcoreMesh(core_axis_name="c", subcore_axis_name="s")


@functools.partial(
    pl.kernel,
    mesh=_mesh,
    out_type=tuple(jax.ShapeDtypeStruct((BATCH, QDIM), jnp.float32)
                   for _ in range(NQ)),
    compiler_params=pltpu.CompilerParams(use_tc_tiling_on_sc=False),
    scratch_types=[
        pltpu.VMEM((N_CHUNK, CHUNK), jnp.int32),
        tuple(pltpu.VMEM((B_PER_W, QDIM), jnp.float32) for _ in range(NQ)),
        pltpu.SemaphoreType.DMA,
    ],
)
def _gather_kernel(t0, t1, t2, t3, idx_hbm, out0, out1, out2, out3,
                   idx_v, rows_v, sem):
    tables = (t0, t1, t2, t3)
    outs = (out0, out1, out2, out3)
    wid = lax.axis_index("s") * NC + lax.axis_index("c")
    base = wid * B_PER_W
    pltpu.sync_copy(idx_hbm.at[wid], idx_v)
    copies = []
    for j in range(N_CHUNK):
        for k in range(NQ):
            copies.append(
                pltpu.async_copy(
                    tables[k].at[idx_v.at[j]],
                    rows_v[k].at[pl.ds(j * CHUNK, CHUNK), :],
                    sem,
                )
            )
    for c in copies:
        c.wait()
    for k in range(NQ):
        pltpu.sync_copy(rows_v[k], outs[k].at[pl.ds(base, B_PER_W)])


def kernel(pt_id, theta_h_weight):
    parts = [theta_h_weight[:, k * QDIM:(k + 1) * QDIM] for k in range(NQ)]
    idx = jnp.clip(pt_id.astype(jnp.int32), 0, MAX_PT - 1)
    outs = _gather_kernel(*parts, idx.reshape(NW, N_CHUNK, CHUNK))
    return jnp.concatenate(outs, axis=1)
